# Initial kernel scaffold; baseline (speedup 1.0000x reference)
import functools

import jax
import jax.numpy as jnp
from jax import lax
from jax.experimental import pallas as pl
from jax.experimental.pallas import tpu as pltpu
from jax.experimental.pallas import tpu_sc as plsc

N = 10000
D = 128
K = 4
E = 320000

NC = 2
NS = 16
NW = NC * NS

CHUNK = 128
CHUNKS_PER_W = 320
EDGES_PER_W = CHUNK * CHUNKS_PER_W
E_PAD = NW * EDGES_PER_W

N_ACC = 10240
ROWS_PER_TILE = N_ACC // NS
MM_BLK = 400


def _mm_body(x_ref, w_ref, o_ref):
    o_ref[0] = jnp.dot(x_ref[...], w_ref[0], preferred_element_type=jnp.float32)


def _matmul(x, W):
    return pl.pallas_call(
        _mm_body,
        grid=(K, N // MM_BLK),
        in_specs=[
            pl.BlockSpec((MM_BLK, D), lambda k, i: (i, 0)),
            pl.BlockSpec((1, D, D), lambda k, i: (k, 0, 0)),
        ],
        out_specs=pl.BlockSpec((1, MM_BLK, D), lambda k, i: (k, i, 0)),
        out_shape=jax.ShapeDtypeStruct((K, N, D), jnp.float32),
    )(x, W)


def _comb_body(p_ref, b_ref, o_ref):
    o_ref[...] = p_ref[0] + p_ref[1] + b_ref[...]


def _combine(partials, bias):
    return pl.pallas_call(
        _comb_body,
        grid=(N // MM_BLK,),
        in_specs=[
            pl.BlockSpec((2, MM_BLK, D), lambda i: (0, i, 0)),
            pl.BlockSpec((1, D), lambda i: (0, 0)),
        ],
        out_specs=pl.BlockSpec((MM_BLK, D), lambda i: (i, 0)),
        out_shape=jax.ShapeDtypeStruct((N, D), jnp.float32),
    )(partials, bias.reshape(1, D))


def _sc_body(xw_hbm, cols_hbm, rows_hbm, vals_hbm, out_hbm,
             idx_v, row_v, vals_v, buf_v, acc_s, sem):
    cid = lax.axis_index("c")
    sid = lax.axis_index("s")
    wid = cid * NS + sid

    zeros16 = jnp.zeros((16,), jnp.float32)

    def zrow(i, _):
        for j in range(D // 16):
            buf_v[i, pl.ds(j * 16, 16)] = zeros16
        return 0

    lax.fori_loop(0, CHUNK, zrow, 0)
    for b in range(ROWS_PER_TILE // CHUNK):
        pltpu.sync_copy(buf_v, acc_s.at[pl.ds(sid * ROWS_PER_TILE + b * CHUNK, CHUNK)])
    plsc.subcore_barrier()

    lane = lax.iota(jnp.int32, 16)
    base_w = wid * EDGES_PER_W

    def chunk_body(c, _):
        base = base_w + c * CHUNK
        pltpu.sync_copy(cols_hbm.at[pl.ds(base, CHUNK)], idx_v)
        pltpu.sync_copy(rows_hbm.at[pl.ds(base, CHUNK)], row_v)
        pltpu.sync_copy(vals_hbm.at[pl.ds(base, CHUNK)], vals_v)
        pltpu.async_copy(xw_hbm.at[idx_v], buf_v, sem).wait()

        def edge_body(e, _):
            eidx = jnp.full((16,), e, jnp.int32)
            vv = plsc.load_gather(vals_v, [eidx])
            for j in range(D // 16):
                colidx = lane + j * 16
                r = plsc.load_gather(buf_v, [eidx, colidx])
                plsc.store_scatter(buf_v, [eidx, colidx], r * vv)
            return 0

        lax.fori_loop(0, CHUNK, edge_body, 0)
        pltpu.sync_copy(buf_v, acc_s.at[row_v], add=True)
        return 0

    lax.fori_loop(0, CHUNKS_PER_W, chunk_body, 0)
    plsc.subcore_barrier()

    pltpu.sync_copy(acc_s.at[pl.ds(sid * ROWS_PER_TILE, ROWS_PER_TILE)],
                    out_hbm.at[cid, pl.ds(sid * ROWS_PER_TILE, ROWS_PER_TILE)])


_sc_agg = functools.partial(
    pl.kernel,
    out_type=jax.ShapeDtypeStruct((NC, N_ACC, D), jnp.float32),
    mesh=plsc.VectorSubcoreMesh(core_axis_name="c", subcore_axis_name="s"),
    scratch_types=[
        pltpu.VMEM((CHUNK,), jnp.int32),
        pltpu.VMEM((CHUNK,), jnp.int32),
        pltpu.VMEM((CHUNK,), jnp.float32),
        pltpu.VMEM((CHUNK, D), jnp.float32),
        pltpu.VMEM_SHARED((N_ACC, D), jnp.float32),
        pltpu.SemaphoreType.DMA,
    ],
)(_sc_body)


def kernel(x, adj_indices, adj_values, W0, W1, W2, W3, bias):
    W = jnp.stack([W0, W1, W2, W3])
    xw = _matmul(x, W).reshape(K * N, D)

    ai = adj_indices.astype(jnp.int32)
    rows = ai[:, 0, :].reshape(-1)
    cols = (ai[:, 1, :] + (jnp.arange(K, dtype=jnp.int32) * N)[:, None]).reshape(-1)
    vals = adj_values.reshape(-1)
    pad = E_PAD - K * E
    rows = jnp.concatenate([rows, jnp.zeros((pad,), jnp.int32)])
    cols = jnp.concatenate([cols, jnp.zeros((pad,), jnp.int32)])
    vals = jnp.concatenate([vals, jnp.zeros((pad,), jnp.float32)])

    partials = _sc_agg(xw, cols, rows, vals)
    return _combine(partials[:, :N, :], bias)


# speedup vs baseline: 2.1545x; 2.1545x over previous
import functools

import jax
import jax.numpy as jnp
from jax import lax
from jax.experimental import pallas as pl
from jax.experimental.pallas import tpu as pltpu
from jax.experimental.pallas import tpu_sc as plsc

N = 10000
D = 128
K = 4
E = 320000

NC = 2
NS = 16
NW = NC * NS

CHUNK = 128
CHUNKS_PER_W = 320
EDGES_PER_W = CHUNK * CHUNKS_PER_W
E_PAD = NW * EDGES_PER_W

N_ACC = 10240
ROWS_PER_TILE = N_ACC // NS
MM_BLK = 400


def _mm_body(x_ref, w_ref, o_ref):
    o_ref[0] = jnp.dot(x_ref[...], w_ref[0], preferred_element_type=jnp.float32)


def _matmul(x, W):
    return pl.pallas_call(
        _mm_body,
        grid=(K, N // MM_BLK),
        in_specs=[
            pl.BlockSpec((MM_BLK, D), lambda k, i: (i, 0)),
            pl.BlockSpec((1, D, D), lambda k, i: (k, 0, 0)),
        ],
        out_specs=pl.BlockSpec((1, MM_BLK, D), lambda k, i: (k, i, 0)),
        out_shape=jax.ShapeDtypeStruct((K, N, D), jnp.float32),
    )(x, W)


def _comb_body(p_ref, b_ref, o_ref):
    o_ref[...] = p_ref[0] + p_ref[1] + b_ref[...]


def _combine(partials, bias):
    return pl.pallas_call(
        _comb_body,
        grid=(N // MM_BLK,),
        in_specs=[
            pl.BlockSpec((2, MM_BLK, D), lambda i: (0, i, 0)),
            pl.BlockSpec((1, D), lambda i: (0, 0)),
        ],
        out_specs=pl.BlockSpec((MM_BLK, D), lambda i: (i, 0)),
        out_shape=jax.ShapeDtypeStruct((N, D), jnp.float32),
    )(partials, bias.reshape(1, D))


def _sc_body(xw_hbm, cols_hbm, rows_hbm, vals_hbm, out_hbm,
             idx_v, row_v, vals_v, buf_v, acc_s, sem):
    cid = lax.axis_index("c")
    sid = lax.axis_index("s")
    wid = cid * NS + sid

    zeros16 = jnp.zeros((16,), jnp.float32)

    def zrow(i, _):
        for j in range(D // 16):
            buf_v[i, pl.ds(j * 16, 16)] = zeros16
        return 0

    lax.fori_loop(0, CHUNK, zrow, 0)
    for b in range(ROWS_PER_TILE // CHUNK):
        pltpu.sync_copy(buf_v, acc_s.at[pl.ds(sid * ROWS_PER_TILE + b * CHUNK, CHUNK)])
    plsc.subcore_barrier()

    base_w = wid * EDGES_PER_W

    def chunk_body(c, _):
        base = base_w + c * CHUNK
        pltpu.sync_copy(cols_hbm.at[pl.ds(base, CHUNK)], idx_v)
        pltpu.sync_copy(rows_hbm.at[pl.ds(base, CHUNK)], row_v)
        pltpu.sync_copy(vals_hbm.at[pl.ds(base, CHUNK)], vals_v)
        pltpu.async_copy(xw_hbm.at[idx_v], buf_v, sem).wait()

        def group_body(g, _):
            vv16 = vals_v[pl.ds(g * 16, 16)]
            for r in range(16):
                vv = jnp.full((16,), vv16[r])
                e = g * 16 + r
                for j in range(D // 16):
                    sl = pl.ds(j * 16, 16)
                    buf_v[e, sl] = buf_v[e, sl] * vv
            return 0

        lax.fori_loop(0, CHUNK // 16, group_body, 0)
        pltpu.sync_copy(buf_v, acc_s.at[row_v], add=True)
        return 0

    lax.fori_loop(0, CHUNKS_PER_W, chunk_body, 0)
    plsc.subcore_barrier()

    pltpu.sync_copy(acc_s.at[pl.ds(sid * ROWS_PER_TILE, ROWS_PER_TILE)],
                    out_hbm.at[cid, pl.ds(sid * ROWS_PER_TILE, ROWS_PER_TILE)])


_sc_agg = functools.partial(
    pl.kernel,
    out_type=jax.ShapeDtypeStruct((NC, N_ACC, D), jnp.float32),
    mesh=plsc.VectorSubcoreMesh(core_axis_name="c", subcore_axis_name="s"),
    scratch_types=[
        pltpu.VMEM((CHUNK,), jnp.int32),
        pltpu.VMEM((CHUNK,), jnp.int32),
        pltpu.VMEM((CHUNK,), jnp.float32),
        pltpu.VMEM((CHUNK, D), jnp.float32),
        pltpu.VMEM_SHARED((N_ACC, D), jnp.float32),
        pltpu.SemaphoreType.DMA,
    ],
)(_sc_body)


def kernel(x, adj_indices, adj_values, W0, W1, W2, W3, bias):
    W = jnp.stack([W0, W1, W2, W3])
    xw = _matmul(x, W).reshape(K * N, D)

    ai = adj_indices.astype(jnp.int32)
    rows = ai[:, 0, :].reshape(-1)
    cols = (ai[:, 1, :] + (jnp.arange(K, dtype=jnp.int32) * N)[:, None]).reshape(-1)
    vals = adj_values.reshape(-1)
    pad = E_PAD - K * E
    rows = jnp.concatenate([rows, jnp.zeros((pad,), jnp.int32)])
    cols = jnp.concatenate([cols, jnp.zeros((pad,), jnp.int32)])
    vals = jnp.concatenate([vals, jnp.zeros((pad,), jnp.float32)])

    partials = _sc_agg(xw, cols, rows, vals)
    return _combine(partials[:, :N, :], bias)


# speedup vs baseline: 2.9883x; 1.3870x over previous
import functools

import jax
import jax.numpy as jnp
from jax import lax
from jax.experimental import pallas as pl
from jax.experimental.pallas import tpu as pltpu
from jax.experimental.pallas import tpu_sc as plsc

N = 10000
D = 128
K = 4
E = 320000

NC = 2
NS = 16
NW = NC * NS

CHUNK = 128
CHUNKS_PER_W = 320
EDGES_PER_W = CHUNK * CHUNKS_PER_W
E_PAD = NW * EDGES_PER_W
SUP = 32
SUPE = SUP * CHUNK
N_SUPER = CHUNKS_PER_W // SUP

N_ACC = 10240
ROWS_PER_TILE = N_ACC // NS
MM_BLK = 400


def _mm_body(x_ref, w_ref, o_ref):
    o_ref[0] = jnp.dot(x_ref[...], w_ref[0], preferred_element_type=jnp.float32)


def _matmul(x, W):
    return pl.pallas_call(
        _mm_body,
        grid=(K, N // MM_BLK),
        in_specs=[
            pl.BlockSpec((MM_BLK, D), lambda k, i: (i, 0)),
            pl.BlockSpec((1, D, D), lambda k, i: (k, 0, 0)),
        ],
        out_specs=pl.BlockSpec((1, MM_BLK, D), lambda k, i: (k, i, 0)),
        out_shape=jax.ShapeDtypeStruct((K, N, D), jnp.float32),
    )(x, W)


def _comb_body(p_ref, b_ref, o_ref):
    o_ref[...] = p_ref[0] + p_ref[1] + b_ref[...]


def _combine(partials, bias):
    return pl.pallas_call(
        _comb_body,
        grid=(N // MM_BLK,),
        in_specs=[
            pl.BlockSpec((2, MM_BLK, D), lambda i: (0, i, 0)),
            pl.BlockSpec((1, D), lambda i: (0, 0)),
        ],
        out_specs=pl.BlockSpec((MM_BLK, D), lambda i: (i, 0)),
        out_shape=jax.ShapeDtypeStruct((N, D), jnp.float32),
    )(partials, bias.reshape(1, D))


def _sc_body(xw_hbm, cols_hbm, rows_hbm, vals_hbm, out_hbm,
             idx_sv, vals_sv, buf0, buf1, row0, row1, acc_s, sem0, sem1):
    cid = lax.axis_index("c")
    sid = lax.axis_index("s")
    wid = cid * NS + sid

    zeros16 = jnp.zeros((16,), jnp.float32)

    def zrow(i, _):
        for j in range(D // 16):
            buf0[i, pl.ds(j * 16, 16)] = zeros16
        return 0

    lax.fori_loop(0, CHUNK, zrow, 0)
    for b in range(ROWS_PER_TILE // CHUNK):
        pltpu.sync_copy(buf0, acc_s.at[pl.ds(sid * ROWS_PER_TILE + b * CHUNK, CHUNK)])
    plsc.subcore_barrier()

    base_w = wid * EDGES_PER_W

    def gather_issue(base, cc, buf, row, sem):
        off = pl.multiple_of(cc * CHUNK, CHUNK)
        aoff = pl.multiple_of(base + cc * CHUNK, CHUNK)
        pltpu.async_copy(xw_hbm.at[idx_sv.at[pl.ds(off, CHUNK)]], buf, sem)
        pltpu.async_copy(rows_hbm.at[pl.ds(aoff, CHUNK)], row, sem)

    def gather_wait(buf, row, sem):
        pltpu.make_async_copy(xw_hbm.at[pl.ds(0, CHUNK)], buf, sem).wait()
        pltpu.make_async_copy(rows_hbm.at[pl.ds(0, CHUNK)], row, sem).wait()

    def scale_chunk(buf, cc):
        def group_body(g, _):
            off = pl.multiple_of(cc * CHUNK + g * 16, 16)
            vv16 = vals_sv[pl.ds(off, 16)]
            for r in range(16):
                vv = jnp.full((16,), vv16[r])
                e = g * 16 + r
                for j in range(D // 16):
                    sl = pl.ds(j * 16, 16)
                    buf[e, sl] = buf[e, sl] * vv
            return 0

        lax.fori_loop(0, CHUNK // 16, group_body, 0)

    def scatter_chunk(buf, row):
        pltpu.sync_copy(buf, acc_s.at[row], add=True)

    def super_body(s, _):
        base = base_w + s * SUPE
        pltpu.sync_copy(cols_hbm.at[pl.ds(base, SUPE)], idx_sv)
        pltpu.sync_copy(vals_hbm.at[pl.ds(base, SUPE)], vals_sv)

        gather_issue(base, 0, buf0, row0, sem0)

        def pair_body(p, _):
            a = 2 * p
            gather_issue(base, a + 1, buf1, row1, sem1)
            gather_wait(buf0, row0, sem0)
            scale_chunk(buf0, a)
            scatter_chunk(buf0, row0)

            @pl.when(p < SUP // 2 - 1)
            def _():
                gather_issue(base, a + 2, buf0, row0, sem0)

            gather_wait(buf1, row1, sem1)
            scale_chunk(buf1, a + 1)
            scatter_chunk(buf1, row1)
            return 0

        lax.fori_loop(0, SUP // 2, pair_body, 0)
        return 0

    lax.fori_loop(0, N_SUPER, super_body, 0)
    plsc.subcore_barrier()

    pltpu.sync_copy(acc_s.at[pl.ds(sid * ROWS_PER_TILE, ROWS_PER_TILE)],
                    out_hbm.at[cid, pl.ds(sid * ROWS_PER_TILE, ROWS_PER_TILE)])


_sc_agg = functools.partial(
    pl.kernel,
    out_type=jax.ShapeDtypeStruct((NC, N_ACC, D), jnp.float32),
    mesh=plsc.VectorSubcoreMesh(core_axis_name="c", subcore_axis_name="s"),
    scratch_types=[
        pltpu.VMEM((SUPE,), jnp.int32),
        pltpu.VMEM((SUPE,), jnp.float32),
        pltpu.VMEM((CHUNK, D), jnp.float32),
        pltpu.VMEM((CHUNK, D), jnp.float32),
        pltpu.VMEM((CHUNK,), jnp.int32),
        pltpu.VMEM((CHUNK,), jnp.int32),
        pltpu.VMEM_SHARED((N_ACC, D), jnp.float32),
        pltpu.SemaphoreType.DMA,
        pltpu.SemaphoreType.DMA,
    ],
)(_sc_body)


def kernel(x, adj_indices, adj_values, W0, W1, W2, W3, bias):
    W = jnp.stack([W0, W1, W2, W3])
    xw = _matmul(x, W).reshape(K * N, D)

    ai = adj_indices.astype(jnp.int32)
    rows = ai[:, 0, :].reshape(-1)
    cols = (ai[:, 1, :] + (jnp.arange(K, dtype=jnp.int32) * N)[:, None]).reshape(-1)
    vals = adj_values.reshape(-1)
    pad = E_PAD - K * E
    rows = jnp.concatenate([rows, jnp.zeros((pad,), jnp.int32)])
    cols = jnp.concatenate([cols, jnp.zeros((pad,), jnp.int32)])
    vals = jnp.concatenate([vals, jnp.zeros((pad,), jnp.float32)])

    partials = _sc_agg(xw, cols, rows, vals)
    return _combine(partials[:, :N, :], bias)


# speedup vs baseline: 8.6636x; 2.8992x over previous
import functools

import jax
import jax.numpy as jnp
from jax import lax
from jax.experimental import pallas as pl
from jax.experimental.pallas import tpu as pltpu
from jax.experimental.pallas import tpu_sc as plsc

N = 10000
D = 128
K = 4
E = 320000

NC = 2
NS = 16
NW = NC * NS

CHUNK = 128
CHUNKS_PER_W = 320
EDGES_PER_W = CHUNK * CHUNKS_PER_W
E_PAD = NW * EDGES_PER_W
SUP = 32
SUPE = SUP * CHUNK
N_SUPER = CHUNKS_PER_W // SUP

N_ACC = 10240
ROWS_PER_TILE = N_ACC // NS
MM_BLK = 400


def _mm_body(x_ref, w_ref, o_ref):
    o_ref[0] = jnp.dot(x_ref[...], w_ref[0], preferred_element_type=jnp.float32)


def _matmul(x, W):
    return pl.pallas_call(
        _mm_body,
        grid=(K, N // MM_BLK),
        in_specs=[
            pl.BlockSpec((MM_BLK, D), lambda k, i: (i, 0)),
            pl.BlockSpec((1, D, D), lambda k, i: (k, 0, 0)),
        ],
        out_specs=pl.BlockSpec((1, MM_BLK, D), lambda k, i: (k, i, 0)),
        out_shape=jax.ShapeDtypeStruct((K, N, D), jnp.float32),
    )(x, W)


def _comb_body(p_ref, b_ref, o_ref):
    o_ref[...] = p_ref[0] + p_ref[1] + b_ref[...]


def _combine(partials, bias):
    return pl.pallas_call(
        _comb_body,
        grid=(N // MM_BLK,),
        in_specs=[
            pl.BlockSpec((2, MM_BLK, D), lambda i: (0, i, 0)),
            pl.BlockSpec((1, D), lambda i: (0, 0)),
        ],
        out_specs=pl.BlockSpec((MM_BLK, D), lambda i: (i, 0)),
        out_shape=jax.ShapeDtypeStruct((N, D), jnp.float32),
    )(partials, bias.reshape(1, D))


def _sc_body(xw_hbm, cols_hbm, rows_hbm, vals_hbm, out_hbm,
             idx_sv, vals_sv, buf0, buf1, row0, row1, acc_s, sem0, sem1):
    cid = lax.axis_index("c")
    sid = lax.axis_index("s")
    wid = cid * NS + sid

    zeros16 = jnp.zeros((16,), jnp.float32)

    def zrow(i, _):
        for j in range(D // 16):
            buf0[i, pl.ds(j * 16, 16)] = zeros16
        return 0

    lax.fori_loop(0, CHUNK, zrow, 0)
    for b in range(ROWS_PER_TILE // CHUNK):
        pltpu.sync_copy(buf0, acc_s.at[pl.ds(sid * ROWS_PER_TILE + b * CHUNK, CHUNK)])
    plsc.subcore_barrier()

    base_w = wid * EDGES_PER_W

    def gather_issue(base, cc, buf, row, sem):
        off = pl.multiple_of(cc * CHUNK, CHUNK)
        aoff = pl.multiple_of(base + cc * CHUNK, CHUNK)
        pltpu.async_copy(xw_hbm.at[idx_sv.at[pl.ds(off, CHUNK)]], buf, sem)
        pltpu.async_copy(rows_hbm.at[pl.ds(aoff, CHUNK)], row, sem)

    def gather_wait(buf, row, sem):
        pltpu.make_async_copy(xw_hbm.at[pl.ds(0, CHUNK)], buf, sem).wait()
        pltpu.make_async_copy(rows_hbm.at[pl.ds(0, CHUNK)], row, sem).wait()

    def scale_chunk(buf, cc):
        def group_body(g, _):
            off = pl.multiple_of(cc * CHUNK + g * 16, 16)
            vv16 = vals_sv[pl.ds(off, 16)]
            for r in range(16):
                vv = jnp.full((16,), vv16[r])
                e = g * 16 + r
                for j in range(D // 16):
                    sl = pl.ds(j * 16, 16)
                    buf[e, sl] = buf[e, sl] * vv
            return 0

        lax.fori_loop(0, CHUNK // 16, group_body, 0)

    def scatter_chunk(buf, row):
        pltpu.sync_copy(buf, acc_s.at[row], add=True)

    def super_body(s, _):
        base = base_w + s * SUPE
        pltpu.sync_copy(cols_hbm.at[pl.ds(base, SUPE)], idx_sv)
        pltpu.sync_copy(vals_hbm.at[pl.ds(base, SUPE)], vals_sv)

        gather_issue(base, 0, buf0, row0, sem0)

        def pair_body(p, _):
            a = 2 * p
            gather_issue(base, a + 1, buf1, row1, sem1)
            gather_wait(buf0, row0, sem0)
            scale_chunk(buf0, a)
            scatter_chunk(buf0, row0)

            @pl.when(p < SUP // 2 - 1)
            def _():
                gather_issue(base, a + 2, buf0, row0, sem0)

            gather_wait(buf1, row1, sem1)
            scale_chunk(buf1, a + 1)
            scatter_chunk(buf1, row1)
            return 0

        lax.fori_loop(0, SUP // 2, pair_body, 0)
        return 0

    lax.fori_loop(0, N_SUPER, super_body, 0)
    plsc.subcore_barrier()

    pltpu.sync_copy(acc_s.at[pl.ds(sid * ROWS_PER_TILE, ROWS_PER_TILE)],
                    out_hbm.at[cid, pl.ds(sid * ROWS_PER_TILE, ROWS_PER_TILE)])


_sc_agg = functools.partial(
    pl.kernel,
    out_type=jax.ShapeDtypeStruct((NC, N_ACC, D), jnp.float32),
    mesh=plsc.VectorSubcoreMesh(core_axis_name="c", subcore_axis_name="s"),
    scratch_types=[
        pltpu.VMEM((SUPE,), jnp.int32),
        pltpu.VMEM((SUPE,), jnp.float32),
        pltpu.VMEM((CHUNK, D), jnp.float32),
        pltpu.VMEM((CHUNK, D), jnp.float32),
        pltpu.VMEM((CHUNK,), jnp.int32),
        pltpu.VMEM((CHUNK,), jnp.int32),
        pltpu.VMEM_SHARED((N_ACC, D), jnp.float32),
        pltpu.SemaphoreType.DMA,
        pltpu.SemaphoreType.DMA,
    ],
)(_sc_body)


def kernel(x, adj_indices, adj_values, W0, W1, W2, W3, bias):
    W = jnp.stack([W0, W1, W2, W3])
    xw = _matmul(x, W).reshape(K * N, D)

    ai = adj_indices.astype(jnp.int32)
    rows = ai[:, 0, :].reshape(-1)
    cols = (ai[:, 1, :] + (jnp.arange(K, dtype=jnp.int32) * N)[:, None]).reshape(-1)
    vals = adj_values.reshape(-1)

    ew = K * E // NW
    pw = EDGES_PER_W - ew
    pad_idx = jnp.broadcast_to(jnp.arange(pw, dtype=jnp.int32) % N, (NW, pw))
    rows = jnp.concatenate([rows.reshape(NW, ew), pad_idx], axis=1).reshape(-1)
    cols = jnp.concatenate([cols.reshape(NW, ew), pad_idx], axis=1).reshape(-1)
    vals = jnp.concatenate([vals.reshape(NW, ew),
                            jnp.zeros((NW, pw), jnp.float32)], axis=1).reshape(-1)

    partials = _sc_agg(xw, cols, rows, vals)
    return _combine(partials[:, :N, :], bias)


# speedup vs baseline: 9.5454x; 1.1018x over previous
import functools

import jax
import jax.numpy as jnp
from jax import lax
from jax.experimental import pallas as pl
from jax.experimental.pallas import tpu as pltpu
from jax.experimental.pallas import tpu_sc as plsc

N = 10000
D = 128
K = 4
E = 320000

NC = 2
NS = 16
NW = NC * NS

CHUNK = 96
NCH = 420
SUP = 60
SUPE = SUP * CHUNK
N_SUPER = NCH // SUP
EDGES_PER_W = CHUNK * NCH
E_PAD = NW * EDGES_PER_W

N_ACC = 10112
ROWS_PER_TILE = N_ACC // NS
MM_BLK = 400


def _mm_body(x_ref, w_ref, o_ref):
    o_ref[0] = jnp.dot(x_ref[...], w_ref[0], preferred_element_type=jnp.float32)


def _matmul(x, W):
    return pl.pallas_call(
        _mm_body,
        grid=(K, N // MM_BLK),
        in_specs=[
            pl.BlockSpec((MM_BLK, D), lambda k, i: (i, 0)),
            pl.BlockSpec((1, D, D), lambda k, i: (k, 0, 0)),
        ],
        out_specs=pl.BlockSpec((1, MM_BLK, D), lambda k, i: (k, i, 0)),
        out_shape=jax.ShapeDtypeStruct((K, N, D), jnp.float32),
    )(x, W)


def _comb_body(p_ref, b_ref, o_ref):
    o_ref[...] = p_ref[0] + p_ref[1] + b_ref[...]


def _combine(partials, bias):
    return pl.pallas_call(
        _comb_body,
        grid=(N // MM_BLK,),
        in_specs=[
            pl.BlockSpec((2, MM_BLK, D), lambda i: (0, i, 0)),
            pl.BlockSpec((1, D), lambda i: (0, 0)),
        ],
        out_specs=pl.BlockSpec((MM_BLK, D), lambda i: (i, 0)),
        out_shape=jax.ShapeDtypeStruct((N, D), jnp.float32),
    )(partials, bias.reshape(1, D))


def _sc_body(xw_hbm, cols_hbm, rows_hbm, vals_hbm, out_hbm,
             idx_sv, vals_sv, row0, row1, row2, buf0, buf1, buf2, acc_s,
             sg0, sg1, sg2, ss0, ss1, ss2):
    cid = lax.axis_index("c")
    sid = lax.axis_index("s")
    wid = cid * NS + sid

    rows_b = (row0, row1, row2)
    bufs = (buf0, buf1, buf2)
    sg = (sg0, sg1, sg2)
    ss = (ss0, ss1, ss2)

    zeros16 = jnp.zeros((16,), jnp.float32)

    def zrow(i, _):
        for j in range(D // 16):
            buf0[i, pl.ds(j * 16, 16)] = zeros16
        return 0

    lax.fori_loop(0, CHUNK, zrow, 0)
    zbase = sid * ROWS_PER_TILE
    for b in range(ROWS_PER_TILE // CHUNK):
        pltpu.sync_copy(buf0, acc_s.at[pl.ds(zbase + b * CHUNK, CHUNK)])
    rem = ROWS_PER_TILE % CHUNK
    if rem:
        pltpu.sync_copy(buf0.at[pl.ds(0, rem)],
                        acc_s.at[pl.ds(zbase + ROWS_PER_TILE - rem, rem)])
    plsc.subcore_barrier()

    base_w = wid * EDGES_PER_W

    def issue_gr(base, c, b):
        off = pl.multiple_of(c * CHUNK, 8)
        aoff = pl.multiple_of(base + c * CHUNK, 8)
        pltpu.async_copy(xw_hbm.at[idx_sv.at[pl.ds(off, CHUNK)]], bufs[b], sg[b])
        pltpu.async_copy(rows_hbm.at[pl.ds(aoff, CHUNK)], rows_b[b], sg[b])

    def wait_gr(b):
        pltpu.make_async_copy(xw_hbm.at[pl.ds(0, CHUNK)], bufs[b], sg[b]).wait()
        pltpu.make_async_copy(rows_hbm.at[pl.ds(0, CHUNK)], rows_b[b], sg[b]).wait()

    def issue_s(b):
        pltpu.async_copy(bufs[b], acc_s.at[rows_b[b]], ss[b], add=True)

    def wait_s(b):
        pltpu.make_async_copy(bufs[b], acc_s.at[pl.ds(0, CHUNK)], ss[b]).wait()

    def scale(b, c):
        buf = bufs[b]

        def group_body(g, _):
            off = pl.multiple_of(c * CHUNK + g * 16, 16)
            vv16 = vals_sv[pl.ds(off, 16)]
            for r in range(16):
                vv = jnp.full((16,), vv16[r])
                e = g * 16 + r
                for j in range(D // 16):
                    sl = pl.ds(j * 16, 16)
                    buf[e, sl] = buf[e, sl] * vv
            return 0

        lax.fori_loop(0, CHUNK // 16, group_body, 0)

    def chunk_op(base, c, b, first=False, last=False):
        if not first:
            wait_s((b + 1) % 3)
        if not last:
            issue_gr(base, c + 1, (b + 1) % 3)
        wait_gr(b)
        scale(b, c)
        issue_s(b)

    def super_body(s, _):
        base = base_w + s * SUPE
        boff = pl.multiple_of(base, 8)
        pltpu.sync_copy(cols_hbm.at[pl.ds(boff, SUPE)], idx_sv)
        pltpu.sync_copy(vals_hbm.at[pl.ds(boff, SUPE)], vals_sv)

        issue_gr(base, 0, 0)
        chunk_op(base, 0, 0, first=True)
        chunk_op(base, 1, 1, first=True)

        def tri_body(t, _):
            c0 = 3 * t + 2
            for u in range(3):
                chunk_op(base, c0 + u, (2 + u) % 3)
            return 0

        lax.fori_loop(0, (SUP - 3) // 3, tri_body, 0)

        chunk_op(base, SUP - 1, (SUP - 1) % 3, last=True)
        wait_s((SUP - 2) % 3)
        wait_s((SUP - 1) % 3)
        return 0

    lax.fori_loop(0, N_SUPER, super_body, 0)
    plsc.subcore_barrier()

    pltpu.sync_copy(acc_s.at[pl.ds(sid * ROWS_PER_TILE, ROWS_PER_TILE)],
                    out_hbm.at[cid, pl.ds(sid * ROWS_PER_TILE, ROWS_PER_TILE)])


_sc_agg = functools.partial(
    pl.kernel,
    out_type=jax.ShapeDtypeStruct((NC, N_ACC, D), jnp.float32),
    mesh=plsc.VectorSubcoreMesh(core_axis_name="c", subcore_axis_name="s"),
    scratch_types=[
        pltpu.VMEM((SUPE,), jnp.int32),
        pltpu.VMEM((SUPE,), jnp.float32),
        pltpu.VMEM((CHUNK,), jnp.int32),
        pltpu.VMEM((CHUNK,), jnp.int32),
        pltpu.VMEM((CHUNK,), jnp.int32),
        pltpu.VMEM((CHUNK, D), jnp.float32),
        pltpu.VMEM((CHUNK, D), jnp.float32),
        pltpu.VMEM((CHUNK, D), jnp.float32),
        pltpu.VMEM_SHARED((N_ACC, D), jnp.float32),
    ] + [pltpu.SemaphoreType.DMA] * 6,
)(_sc_body)


def kernel(x, adj_indices, adj_values, W0, W1, W2, W3, bias):
    W = jnp.stack([W0, W1, W2, W3])
    xw = _matmul(x, W).reshape(K * N, D)

    ai = adj_indices.astype(jnp.int32)
    rows = ai[:, 0, :].reshape(-1)
    cols = (ai[:, 1, :] + (jnp.arange(K, dtype=jnp.int32) * N)[:, None]).reshape(-1)
    vals = adj_values.reshape(-1)

    ew = K * E // NW
    pw = EDGES_PER_W - ew
    pad_idx = jnp.broadcast_to(jnp.arange(pw, dtype=jnp.int32) % N, (NW, pw))
    rows = jnp.concatenate([rows.reshape(NW, ew), pad_idx], axis=1).reshape(-1)
    cols = jnp.concatenate([cols.reshape(NW, ew), pad_idx], axis=1).reshape(-1)
    vals = jnp.concatenate([vals.reshape(NW, ew),
                            jnp.zeros((NW, pw), jnp.float32)], axis=1).reshape(-1)

    partials = _sc_agg(xw, cols, rows, vals)
    return _combine(partials[:, :N, :], bias)


# speedup vs baseline: 10.1762x; 1.0661x over previous
import functools

import jax
import jax.numpy as jnp
from jax import lax
from jax.experimental import pallas as pl
from jax.experimental.pallas import tpu as pltpu
from jax.experimental.pallas import tpu_sc as plsc

N = 10000
D = 128
K = 4
E = 320000

NC = 2
NS = 16
NW = NC * NS

CHUNK = 80
NCH = 500
SUP = 100
SUPE = SUP * CHUNK
N_SUPER = NCH // SUP
EDGES_PER_W = CHUNK * NCH

N_ACC = 10112
ROWS_PER_TILE = N_ACC // NS
MM_BLK = 400


def _mm_body(x_ref, w_ref, o_ref):
    o_ref[0] = jnp.dot(x_ref[...], w_ref[0], preferred_element_type=jnp.float32)


def _matmul(x, W):
    return pl.pallas_call(
        _mm_body,
        grid=(K, N // MM_BLK),
        in_specs=[
            pl.BlockSpec((MM_BLK, D), lambda k, i: (i, 0)),
            pl.BlockSpec((1, D, D), lambda k, i: (k, 0, 0)),
        ],
        out_specs=pl.BlockSpec((1, MM_BLK, D), lambda k, i: (k, i, 0)),
        out_shape=jax.ShapeDtypeStruct((K, N, D), jnp.float32),
    )(x, W)


def _comb_body(p_ref, b_ref, o_ref):
    o_ref[...] = p_ref[0] + p_ref[1] + b_ref[...]


def _combine(partials, bias):
    return pl.pallas_call(
        _comb_body,
        grid=(N // MM_BLK,),
        in_specs=[
            pl.BlockSpec((2, MM_BLK, D), lambda i: (0, i, 0)),
            pl.BlockSpec((1, D), lambda i: (0, 0)),
        ],
        out_specs=pl.BlockSpec((MM_BLK, D), lambda i: (i, 0)),
        out_shape=jax.ShapeDtypeStruct((N, D), jnp.float32),
    )(partials, bias.reshape(1, D))


def _sc_body(xw_hbm, cols_hbm, rows_hbm, vals_hbm, out_hbm,
             idx_sv, vals_sv, row0, row1, row2, buf0, buf1, buf2, acc_s,
             sg0, sg1, sg2, ss0, ss1, ss2):
    cid = lax.axis_index("c")
    sid = lax.axis_index("s")
    wid = cid * NS + sid

    rows_b = (row0, row1, row2)
    bufs = (buf0, buf1, buf2)
    sg = (sg0, sg1, sg2)
    ss = (ss0, ss1, ss2)

    zeros16 = jnp.zeros((16,), jnp.float32)

    def zrow(i, _):
        for j in range(D // 16):
            buf0[i, pl.ds(j * 16, 16)] = zeros16
        return 0

    lax.fori_loop(0, CHUNK, zrow, 0)
    zbase = sid * ROWS_PER_TILE
    for b in range(ROWS_PER_TILE // CHUNK):
        pltpu.sync_copy(buf0, acc_s.at[pl.ds(zbase + b * CHUNK, CHUNK)])
    rem = ROWS_PER_TILE % CHUNK
    if rem:
        pltpu.sync_copy(buf0.at[pl.ds(0, rem)],
                        acc_s.at[pl.ds(zbase + ROWS_PER_TILE - rem, rem)])
    plsc.subcore_barrier()

    base_w = wid * EDGES_PER_W

    def issue_gr(base, c, b):
        off = pl.multiple_of(c * CHUNK, 8)
        aoff = pl.multiple_of(base + c * CHUNK, 8)
        pltpu.async_copy(xw_hbm.at[idx_sv.at[pl.ds(off, CHUNK)]], bufs[b], sg[b])
        pltpu.async_copy(rows_hbm.at[pl.ds(aoff, CHUNK)], rows_b[b], sg[b])

    def wait_gr(b):
        pltpu.make_async_copy(xw_hbm.at[pl.ds(0, CHUNK)], bufs[b], sg[b]).wait()
        pltpu.make_async_copy(rows_hbm.at[pl.ds(0, CHUNK)], rows_b[b], sg[b]).wait()

    def issue_s(b):
        pltpu.async_copy(bufs[b], acc_s.at[rows_b[b]], ss[b], add=True)

    def wait_s(b):
        pltpu.make_async_copy(bufs[b], acc_s.at[pl.ds(0, CHUNK)], ss[b]).wait()

    def scale(b, c):
        buf = bufs[b]

        def group_body(g, _):
            off = pl.multiple_of(c * CHUNK + g * 16, 16)
            vv16 = vals_sv[pl.ds(off, 16)]
            for r in range(16):
                vv = jnp.full((16,), vv16[r])
                e = g * 16 + r
                for j in range(D // 16):
                    sl = pl.ds(j * 16, 16)
                    buf[e, sl] = buf[e, sl] * vv
            return 0

        lax.fori_loop(0, CHUNK // 16, group_body, 0)

    def chunk_op(base, c, b, first=False, last=False):
        if not first:
            wait_s((b + 1) % 3)
        if not last:
            issue_gr(base, c + 1, (b + 1) % 3)
        wait_gr(b)
        scale(b, c)
        issue_s(b)

    def super_body(s, _):
        base = base_w + s * SUPE
        boff = pl.multiple_of(base, 8)
        pltpu.sync_copy(cols_hbm.at[pl.ds(boff, SUPE)], idx_sv)
        pltpu.sync_copy(vals_hbm.at[pl.ds(boff, SUPE)], vals_sv)

        issue_gr(base, 0, 0)
        chunk_op(base, 0, 0, first=True)
        chunk_op(base, 1, 1, first=True)

        def tri_body(t, _):
            c0 = 3 * t + 2
            for u in range(3):
                chunk_op(base, c0 + u, (2 + u) % 3)
            return 0

        lax.fori_loop(0, (SUP - 4) // 3, tri_body, 0)

        chunk_op(base, SUP - 2, (SUP - 2) % 3)
        chunk_op(base, SUP - 1, (SUP - 1) % 3, last=True)
        wait_s((SUP - 2) % 3)
        wait_s((SUP - 1) % 3)
        return 0

    lax.fori_loop(0, N_SUPER, super_body, 0)
    plsc.subcore_barrier()

    pltpu.sync_copy(acc_s.at[pl.ds(sid * ROWS_PER_TILE, ROWS_PER_TILE)],
                    out_hbm.at[cid, pl.ds(sid * ROWS_PER_TILE, ROWS_PER_TILE)])


_sc_agg = functools.partial(
    pl.kernel,
    out_type=jax.ShapeDtypeStruct((NC, N_ACC, D), jnp.float32),
    mesh=plsc.VectorSubcoreMesh(core_axis_name="c", subcore_axis_name="s"),
    scratch_types=[
        pltpu.VMEM((SUPE,), jnp.int32),
        pltpu.VMEM((SUPE,), jnp.float32),
        pltpu.VMEM((CHUNK,), jnp.int32),
        pltpu.VMEM((CHUNK,), jnp.int32),
        pltpu.VMEM((CHUNK,), jnp.int32),
        pltpu.VMEM((CHUNK, D), jnp.float32),
        pltpu.VMEM((CHUNK, D), jnp.float32),
        pltpu.VMEM((CHUNK, D), jnp.float32),
        pltpu.VMEM_SHARED((N_ACC, D), jnp.float32),
    ] + [pltpu.SemaphoreType.DMA] * 6,
)(_sc_body)


def kernel(x, adj_indices, adj_values, W0, W1, W2, W3, bias):
    W = jnp.stack([W0, W1, W2, W3])
    xw = _matmul(x, W).reshape(K * N, D)

    ai = adj_indices.astype(jnp.int32)
    rows = ai[:, 0, :].reshape(-1)
    cols = (ai[:, 1, :] + (jnp.arange(K, dtype=jnp.int32) * N)[:, None]).reshape(-1)
    vals = adj_values.reshape(-1)

    partials = _sc_agg(xw, cols, rows, vals)
    return _combine(partials[:, :N, :], bias)
